# baseline (speedup 1.0000x reference)
import numpy as np

import jax
import jax.numpy as jnp
from jax.experimental import pallas as pl
from jax.experimental.pallas import tpu as pltpu


_TOPOLOGY = [0, 0, 0, 0, 1, 2, 3, 4, 5, 6, 7, 8, 9, 9, 9, 12, 13, 14, 16,
             17, 18, 19, 20, 21]
_SKELETON_DIST = 2


def _build_edge_topology(topology):
    return [(topology[i], i) for i in range(1, len(topology))]


def _calc_edge_mat(edges):
    n = len(edges)
    mat = [[100000] * n for _ in range(n)]
    for i in range(n):
        mat[i][i] = 0
    for i, a in enumerate(edges):
        for j, b in enumerate(edges):
            link = 0
            for xx in range(2):
                for yy in range(2):
                    if a[xx] == b[yy]:
                        link = 1
            if link:
                mat[i][j] = 1
    for k in range(n):
        for i in range(n):
            for j in range(n):
                if mat[i][k] + mat[k][j] < mat[i][j]:
                    mat[i][j] = mat[i][k] + mat[k][j]
    return mat


def _find_neighbor(edges, d):
    mat = _calc_edge_mat(edges)
    n = len(mat)
    neighbor_list = []
    for i in range(n):
        neighbor_list.append([j for j in range(n) if mat[i][j] <= d])
    g = neighbor_list[0].copy()
    for i in g:
        neighbor_list[i].append(n)
    g.append(n)
    neighbor_list.append(g)
    return neighbor_list


def _build_pool(edges, channels_per_edge):
    edge_num = len(edges) + 1
    degree = [0] * 100
    for e in edges:
        degree[e[0]] += 1
        degree[e[1]] += 1
    seq_list = []

    def find_seq(j, seq):
        if degree[j] > 2 and j != 0:
            seq_list.append(seq)
            seq = []
        if degree[j] == 1:
            seq_list.append(seq)
            return
        for idx, e in enumerate(edges):
            if e[0] == j:
                find_seq(e[1], seq + [idx])

    find_seq(0, [])
    pooling_list = []
    new_edges = []
    for seq in seq_list:
        if len(seq) % 2 == 1:
            pooling_list.append([seq[0]])
            new_edges.append(edges[seq[0]])
            seq = seq[1:]
        for i in range(0, len(seq), 2):
            pooling_list.append([seq[i], seq[i + 1]])
            new_edges.append((edges[seq[i]][0], edges[seq[i + 1]][1]))
    pooling_list.append([edge_num - 1])
    W = np.zeros((len(pooling_list) * channels_per_edge,
                  edge_num * channels_per_edge), dtype=np.float32)
    for i, pair in enumerate(pooling_list):
        for j in pair:
            for c in range(channels_per_edge):
                W[i * channels_per_edge + c, j * channels_per_edge + c] = \
                    1.0 / len(pair)
    return pooling_list, new_edges, W


def _build_mask(neighbor_list, in_cpj, out_cpj):
    n = len(neighbor_list)
    M = np.zeros((n * out_cpj, n * in_cpj), dtype=np.float32)
    for i, nb in enumerate(neighbor_list):
        cols = np.array([k * in_cpj + c for k in nb for c in range(in_cpj)],
                        dtype=np.int64)
        M[i * out_cpj:(i + 1) * out_cpj, cols] = 1.0
    return M


_EDGES = _build_edge_topology(_TOPOLOGY)
_NL0 = _find_neighbor(_EDGES, _SKELETON_DIST)
_MASK0_NP = _build_mask(_NL0, 3, 6)
_PL, _NEW_EDGES, _POOL_W_NP = _build_pool(_EDGES, 6)
_NL1 = _find_neighbor(_NEW_EDGES, _SKELETON_DIST)
_MASK1_NP = _build_mask(_NL1, 6, 12)

_IN0 = _MASK0_NP.shape[1]
_OUT0 = _MASK0_NP.shape[0]
_P6 = _POOL_W_NP.shape[0]
_OUT1 = _MASK1_NP.shape[0]


_TB = 2048


def _fused_body(x_ref, w0_ref, b0_ref, w1_ref, b1_ref,
                m0_ref, pool_ref, m1_ref,
                out1_ref, out2_ref,
                a0_ref, b0p_ref, w1m_ref):
    @pl.when(pl.program_id(0) == 0)
    def _prep():
        w0m = w0_ref[...] * m0_ref[...]
        a0_ref[...] = jax.lax.dot_general(
            pool_ref[...], w0m, (((1,), (0,)), ((), ())),
            preferred_element_type=jnp.float32)
        b0p_ref[...] = jax.lax.dot_general(
            b0_ref[...], pool_ref[...], (((1,), (1,)), ((), ())),
            preferred_element_type=jnp.float32)
        w1m_ref[...] = w1_ref[...] * m1_ref[...]

    x = x_ref[...]
    h = jax.lax.dot_general(
        x, a0_ref[...], (((1,), (1,)), ((), ())),
        preferred_element_type=jnp.float32) + b0p_ref[...]
    o1 = jnp.where(h >= 0, h, 0.2 * h)
    out1_ref[...] = o1
    h2 = jax.lax.dot_general(
        o1, w1m_ref[...], (((1,), (1,)), ((), ())),
        preferred_element_type=jnp.float32) + b1_ref[...]
    out2_ref[...] = jnp.where(h2 >= 0, h2, 0.2 * h2)


def kernel(x, W0, b0, W1, b1):
    batch = x.shape[0]
    b0r = b0.reshape(1, _OUT0)
    b1r = b1.reshape(1, _OUT1)
    mask0 = jnp.asarray(_MASK0_NP)
    pool = jnp.asarray(_POOL_W_NP)
    mask1 = jnp.asarray(_MASK1_NP)

    rep = lambda i: (0, 0)
    out1, out2 = pl.pallas_call(
        _fused_body,
        grid=(batch // _TB,),
        in_specs=[
            pl.BlockSpec((_TB, _IN0), lambda i: (i, 0)),
            pl.BlockSpec((_OUT0, _IN0), rep),
            pl.BlockSpec((1, _OUT0), rep),
            pl.BlockSpec((_OUT1, _P6), rep),
            pl.BlockSpec((1, _OUT1), rep),
            pl.BlockSpec((_OUT0, _IN0), rep),
            pl.BlockSpec((_P6, _OUT0), rep),
            pl.BlockSpec((_OUT1, _P6), rep),
        ],
        out_specs=[
            pl.BlockSpec((_TB, _P6), lambda i: (i, 0)),
            pl.BlockSpec((_TB, _OUT1), lambda i: (i, 0)),
        ],
        out_shape=[
            jax.ShapeDtypeStruct((batch, _P6), jnp.float32),
            jax.ShapeDtypeStruct((batch, _OUT1), jnp.float32),
        ],
        scratch_shapes=[
            pltpu.VMEM((_P6, _IN0), jnp.float32),
            pltpu.VMEM((1, _P6), jnp.float32),
            pltpu.VMEM((_OUT1, _P6), jnp.float32),
        ],
    )(x, W0, b0r, W1, b1r, mask0, pool, mask1)
    return (x, out1, out2)
